# baseline (speedup 1.0000x reference)
import functools

import jax
import jax.numpy as jnp
from jax import lax
from jax.experimental import pallas as pl
from jax.experimental.pallas import tpu as pltpu
from jax.experimental.pallas import tpu_sc as plsc

_NC = 2
_NS = 16
_NW = _NC * _NS
_CHUNK = 128

_F = 4
_SD = 64
_ED = 32


def _sc_gather_body(text_w, obj_w, tidx, oidx, comb,
                    tidx_v, oidx_v, rows_v, obj_v, acc_v,
                    gsem0, gsem1, ssem0, ssem1):
    cpw = oidx.shape[1]
    c = lax.axis_index("c")
    s = lax.axis_index("s")
    w = s * _NC + c
    base0 = w * (cpw * _CHUNK)
    pltpu.sync_copy(tidx.at[w], tidx_v)
    pltpu.sync_copy(oidx.at[w], oidx_v)
    gsems = (gsem0, gsem1)
    ssems = (ssem0, ssem1)

    def start(ci, b):
        for f in range(_F):
            pltpu.async_copy(text_w.at[tidx_v.at[_F * ci + f]],
                             rows_v.at[b, f], gsems[b])
        pltpu.async_copy(obj_w.at[oidx_v.at[ci]], obj_v.at[b], gsems[b])

    def drain_gather(b):
        for f in range(_F):
            pltpu.make_async_copy(text_w.at[tidx_v.at[0]],
                                  rows_v.at[b, f], gsems[b]).wait()
        pltpu.make_async_copy(obj_w.at[oidx_v.at[0]],
                              obj_v.at[b], gsems[b]).wait()

    def dst_text(base):
        return comb.at[pl.ds(base, _CHUNK), pl.ds(0, _SD)]

    def dst_obj(base):
        return comb.at[pl.ds(base, _CHUNK), pl.ds(_SD, _ED)]

    def drain_store(b):
        pltpu.make_async_copy(acc_v.at[b], dst_text(0), ssems[b]).wait()
        pltpu.make_async_copy(obj_v.at[b], dst_obj(0), ssems[b]).wait()

    def do_sum(b):
        def tok_body(t, _):
            for q in range(_SD // 16):
                sl = pl.ds(q * 16, 16)
                acc_v[b, t, sl] = (
                    rows_v[b, 0, t, sl] + rows_v[b, 1, t, sl]
                    + rows_v[b, 2, t, sl] + rows_v[b, 3, t, sl])
            return 0

        lax.fori_loop(0, _CHUNK, tok_body, 0)

    def fire_stores(ci, b):
        base = base0 + ci * _CHUNK
        pltpu.async_copy(acc_v.at[b], dst_text(base), ssems[b])
        pltpu.async_copy(obj_v.at[b], dst_obj(base), ssems[b])

    start(0, 0)

    def outer(ci2, _):
        for b in range(2):
            ci = ci2 * 2 + b
            nb = 1 - b

            @pl.when(ci >= 1)
            def _():
                drain_store(nb)

            @pl.when(ci + 1 < cpw)
            def _():
                start(ci + 1, nb)

            drain_gather(b)
            do_sum(b)
            fire_stores(ci, b)
        return 0

    lax.fori_loop(0, cpw // 2, outer, 0)
    if cpw % 2 == 1:
        drain_store(1)
        drain_gather(0)
        do_sum(0)
        fire_stores(cpw - 1, 0)
        drain_store(0)
    else:
        drain_store(1)


def _sc_gather(text_w, obj_w, tidx, oidx, n_tokens):
    mesh = plsc.VectorSubcoreMesh(core_axis_name="c", subcore_axis_name="s",
                                  num_cores=_NC, num_subcores=_NS)
    f = pl.kernel(
        _sc_gather_body,
        out_type=jax.ShapeDtypeStruct((n_tokens, 2 * _SD), jnp.float32),
        mesh=mesh,
        scratch_types=[
            pltpu.VMEM(tidx.shape[1:], jnp.int32),
            pltpu.VMEM(oidx.shape[1:], jnp.int32),
            pltpu.VMEM((2, _F, _CHUNK, _SD), jnp.float32),
            pltpu.VMEM((2, _CHUNK, _ED), jnp.float32),
            pltpu.VMEM((2, _CHUNK, _SD), jnp.float32),
            pltpu.SemaphoreType.DMA,
            pltpu.SemaphoreType.DMA,
            pltpu.SemaphoreType.DMA,
            pltpu.SemaphoreType.DMA,
        ],
        compiler_params=pltpu.CompilerParams(use_tc_tiling_on_sc=False),
    )
    return f(text_w, obj_w, tidx, oidx)


def _ln2(x, g, b):
    m = jnp.mean(x, axis=-1, keepdims=True)
    m2 = jnp.mean(x * x, axis=-1, keepdims=True)
    v = m2 - m * m
    return (x - m) * lax.rsqrt(v + 1e-5) * g + b


def _tc_body(tids, oids, fids, masks, tfeat, comb,
             gref, b1r, g1, bl1, w2, b2, g2, bl2, out):
    blk = out.shape[0]
    bf16 = jnp.bfloat16
    f32 = jnp.float32
    tid = tids[0, 0, :]
    oid = oids[0, 0, :]
    fid = fids[0, 0, :]
    it = lax.broadcasted_iota(jnp.int32, (blk, 120), 1)
    ohb = ((it == tid[:, None]) | (it == oid[:, None] + 20)
           | (it == fid[:, None] + 70))
    oh = ohb.astype(bf16)
    cb = comb[...]
    feats = jnp.concatenate(
        [oh, cb[:, :96].astype(bf16), masks[...].astype(bf16),
         tfeat[...].astype(bf16)], axis=1)
    h = jnp.dot(feats, gref[...], preferred_element_type=f32) + b1r[...]
    h = _ln2(h, g1[...], bl1[...])
    h = jnp.maximum(h, 0.0)
    h = jnp.dot(h.astype(bf16), w2[...], preferred_element_type=f32) + b2[...]
    out[...] = _ln2(h, g2[...], bl2[...])


def _tc_body_alias(tids, oids, fids, masks, tfeat, comb,
                   gref, b1r, g1, bl1, w2, b2, g2, bl2, prev, out):
    del prev
    _tc_body(tids, oids, fids, masks, tfeat, comb,
             gref, b1r, g1, bl1, w2, b2, g2, bl2, out)


def _tc_mlp_slice(tids3, oids3, fids3, masks, tfeat, comb_s, weights, tblk,
                  n_total, s, prev):
    ns = comb_s.shape[0]
    gs = ns // tblk
    off = s * gs
    od = weights[-4].shape[1]

    def ids_spec():
        return pl.BlockSpec((1, 1, tblk), lambda i: (i + off, 0, 0))

    def row_spec(d):
        return pl.BlockSpec((tblk, d), lambda i: (i + off, 0))

    def full_spec(shape):
        nd = len(shape)
        return pl.BlockSpec(shape, lambda i: (0,) * nd)

    in_specs = [
        ids_spec(), ids_spec(), ids_spec(),
        row_spec(masks.shape[1]), row_spec(tfeat.shape[1]),
        pl.BlockSpec((tblk, comb_s.shape[1]), lambda i: (i, 0)),
    ] + [full_spec(w.shape) for w in weights]
    ins = [tids3, oids3, fids3, masks, tfeat, comb_s, *weights]
    body = _tc_body
    kwargs = {}
    if prev is not None:
        ins.append(prev)
        in_specs.append(pl.BlockSpec(memory_space=pl.ANY))
        kwargs["input_output_aliases"] = {len(ins) - 1: 0}
        body = _tc_body_alias

    return pl.pallas_call(
        body,
        grid=(gs,),
        in_specs=in_specs,
        out_specs=pl.BlockSpec((tblk, od), lambda i: (i + off, 0)),
        out_shape=jax.ShapeDtypeStruct((n_total, od), jnp.float32),
        **kwargs,
    )(*ins)


def kernel(type_ids, op_ids, fine_ids, obj_hashes, text_hashes, field_masks,
           time_feats, type_W, op_W, fine_W, obj_W, text_W, mask_W, mask_b,
           time_W, time_b, W1, b1, g1, bln1, W2, b2, g2, bln2):
    B, L = type_ids.shape
    N = B * L
    tpw = N // _NW
    cpw = tpw // _CHUNK

    bt = B // _CHUNK
    ns = 5
    tidx = (text_hashes.astype(jnp.int32)
            .reshape(bt, _CHUNK, L, _F)
            .transpose(2, 0, 3, 1)
            .reshape(ns, _NW, _F * cpw // ns, _CHUNK))
    oidx = obj_hashes.astype(jnp.int32).T.reshape(ns, _NW, cpw // ns, _CHUNK)

    combs = [_sc_gather(text_W, obj_W, tidx[si], oidx[si], N // ns)
             for si in range(ns)]

    tblk = 2048
    g = N // tblk
    tids3 = type_ids.astype(jnp.int32).T.reshape(g, 1, tblk)
    oids3 = op_ids.astype(jnp.int32).T.reshape(g, 1, tblk)
    fids3 = fine_ids.astype(jnp.int32).T.reshape(g, 1, tblk)
    masks = (field_masks.astype(jnp.float32).transpose(1, 0, 2)
             .reshape(N, field_masks.shape[-1]))
    tfeat = time_feats.transpose(1, 0, 2).reshape(N, time_feats.shape[-1])
    g_full = jnp.concatenate([
        type_W @ W1[0:32], op_W @ W1[32:64], fine_W @ W1[64:96],
        W1[128:192], W1[96:128],
        mask_W @ W1[192:224], time_W @ W1[224:256]], axis=0)
    b1_eff = (b1 + mask_b @ W1[192:224] + time_b @ W1[224:256]).reshape(1, -1)
    weights = (g_full.astype(jnp.bfloat16), b1_eff,
               g1.reshape(1, -1), bln1.reshape(1, -1),
               W2.astype(jnp.bfloat16), b2.reshape(1, -1),
               g2.reshape(1, -1), bln2.reshape(1, -1))

    out = None
    for si in range(ns):
        out = _tc_mlp_slice(tids3, oids3, fids3, masks, tfeat, combs[si],
                            weights, tblk, N, si, out)
    return out.reshape(L, B, -1).transpose(1, 0, 2)


# speedup vs baseline: 1.0019x; 1.0019x over previous
import functools

import jax
import jax.numpy as jnp
from jax import lax
from jax.experimental import pallas as pl
from jax.experimental.pallas import tpu as pltpu
from jax.experimental.pallas import tpu_sc as plsc

_NC = 2
_NS = 16
_NW = _NC * _NS
_CHUNK = 128

_F = 4
_SD = 64
_ED = 32


def _sc_gather_body(text_w, obj_w, tidx, oidx, comb,
                    tidx_v, oidx_v, rows_v, obj_v, acc_v,
                    gsem0, gsem1, ssem0, ssem1):
    cpw = oidx.shape[1]
    c = lax.axis_index("c")
    s = lax.axis_index("s")
    w = s * _NC + c
    base0 = w * (cpw * _CHUNK)
    pltpu.sync_copy(tidx.at[w], tidx_v)
    pltpu.sync_copy(oidx.at[w], oidx_v)
    gsems = (gsem0, gsem1)
    ssems = (ssem0, ssem1)

    def start(ci, b):
        for f in range(_F):
            pltpu.async_copy(text_w.at[tidx_v.at[_F * ci + f]],
                             rows_v.at[b, f], gsems[b])
        pltpu.async_copy(obj_w.at[oidx_v.at[ci]], obj_v.at[b], gsems[b])

    def drain_gather(b):
        for f in range(_F):
            pltpu.make_async_copy(text_w.at[tidx_v.at[0]],
                                  rows_v.at[b, f], gsems[b]).wait()
        pltpu.make_async_copy(obj_w.at[oidx_v.at[0]],
                              obj_v.at[b], gsems[b]).wait()

    def dst_text(base):
        return comb.at[pl.ds(base, _CHUNK), pl.ds(0, _SD)]

    def dst_obj(base):
        return comb.at[pl.ds(base, _CHUNK), pl.ds(_SD, _ED)]

    def drain_store(b):
        pltpu.make_async_copy(acc_v.at[b], dst_text(0), ssems[b]).wait()
        pltpu.make_async_copy(obj_v.at[b], dst_obj(0), ssems[b]).wait()

    def do_sum(b):
        def tok_body(t, _):
            for q in range(_SD // 16):
                sl = pl.ds(q * 16, 16)
                acc_v[b, t, sl] = (
                    rows_v[b, 0, t, sl] + rows_v[b, 1, t, sl]
                    + rows_v[b, 2, t, sl] + rows_v[b, 3, t, sl])
            return 0

        lax.fori_loop(0, _CHUNK, tok_body, 0)

    def fire_stores(ci, b):
        base = base0 + ci * _CHUNK
        pltpu.async_copy(acc_v.at[b], dst_text(base), ssems[b])
        pltpu.async_copy(obj_v.at[b], dst_obj(base), ssems[b])

    start(0, 0)

    def outer(ci2, _):
        for b in range(2):
            ci = ci2 * 2 + b
            nb = 1 - b

            @pl.when(ci >= 1)
            def _():
                drain_store(nb)

            @pl.when(ci + 1 < cpw)
            def _():
                start(ci + 1, nb)

            drain_gather(b)
            do_sum(b)
            fire_stores(ci, b)
        return 0

    lax.fori_loop(0, cpw // 2, outer, 0)
    if cpw % 2 == 1:
        drain_store(1)
        drain_gather(0)
        do_sum(0)
        fire_stores(cpw - 1, 0)
        drain_store(0)
    else:
        drain_store(1)


def _sc_gather(text_w, obj_w, tidx, oidx, n_tokens):
    mesh = plsc.VectorSubcoreMesh(core_axis_name="c", subcore_axis_name="s",
                                  num_cores=_NC, num_subcores=_NS)
    f = pl.kernel(
        _sc_gather_body,
        out_type=jax.ShapeDtypeStruct((n_tokens, 2 * _SD), jnp.float32),
        mesh=mesh,
        scratch_types=[
            pltpu.VMEM(tidx.shape[1:], jnp.int32),
            pltpu.VMEM(oidx.shape[1:], jnp.int32),
            pltpu.VMEM((2, _F, _CHUNK, _SD), jnp.float32),
            pltpu.VMEM((2, _CHUNK, _ED), jnp.float32),
            pltpu.VMEM((2, _CHUNK, _SD), jnp.float32),
            pltpu.SemaphoreType.DMA,
            pltpu.SemaphoreType.DMA,
            pltpu.SemaphoreType.DMA,
            pltpu.SemaphoreType.DMA,
        ],
        compiler_params=pltpu.CompilerParams(use_tc_tiling_on_sc=False),
    )
    return f(text_w, obj_w, tidx, oidx)


def _ln2(x, g, b):
    m = jnp.mean(x, axis=-1, keepdims=True)
    m2 = jnp.mean(x * x, axis=-1, keepdims=True)
    v = m2 - m * m
    return (x - m) * lax.rsqrt(v + 1e-5) * g + b


def _tc_body(tids, oids, fids, masks, tfeat, comb,
             gref, b1r, g1, bl1, w2, b2, g2, bl2, out):
    blk = out.shape[0]
    bf16 = jnp.bfloat16
    f32 = jnp.float32
    tid = tids[0, 0, :]
    oid = oids[0, 0, :]
    fid = fids[0, 0, :]
    it = lax.broadcasted_iota(jnp.int32, (blk, 120), 1)
    ohb = ((it == tid[:, None]) | (it == oid[:, None] + 20)
           | (it == fid[:, None] + 70))
    oh = ohb.astype(bf16)
    cb = comb[...]
    feats = jnp.concatenate(
        [oh, cb[:, :96].astype(bf16), masks[...].astype(bf16),
         tfeat[...].astype(bf16)], axis=1)
    h = jnp.dot(feats, gref[...], preferred_element_type=f32) + b1r[...]
    h = _ln2(h, g1[...], bl1[...])
    h = jnp.maximum(h, 0.0)
    h = jnp.dot(h.astype(bf16), w2[...], preferred_element_type=f32) + b2[...]
    out[...] = _ln2(h, g2[...], bl2[...])


def _tc_body_alias(tids, oids, fids, masks, tfeat, comb,
                   gref, b1r, g1, bl1, w2, b2, g2, bl2, prev, out):
    del prev
    _tc_body(tids, oids, fids, masks, tfeat, comb,
             gref, b1r, g1, bl1, w2, b2, g2, bl2, out)


def _tc_mlp_slice(tids3, oids3, fids3, masks, tfeat, comb_s, weights, tblk,
                  n_total, s, prev):
    ns = comb_s.shape[0]
    gs = ns // tblk
    off = s * gs
    od = weights[-4].shape[1]

    def ids_spec():
        return pl.BlockSpec((1, 1, tblk), lambda i: (i + off, 0, 0))

    def row_spec(d):
        return pl.BlockSpec((tblk, d), lambda i: (i + off, 0))

    def full_spec(shape):
        nd = len(shape)
        return pl.BlockSpec(shape, lambda i: (0,) * nd)

    in_specs = [
        ids_spec(), ids_spec(), ids_spec(),
        row_spec(masks.shape[1]), row_spec(tfeat.shape[1]),
        pl.BlockSpec((tblk, comb_s.shape[1]), lambda i: (i, 0)),
    ] + [full_spec(w.shape) for w in weights]
    ins = [tids3, oids3, fids3, masks, tfeat, comb_s, *weights]
    body = _tc_body
    kwargs = {}
    if prev is not None:
        ins.append(prev)
        in_specs.append(pl.BlockSpec(memory_space=pl.ANY))
        kwargs["input_output_aliases"] = {len(ins) - 1: 0}
        body = _tc_body_alias

    return pl.pallas_call(
        body,
        grid=(gs,),
        in_specs=in_specs,
        out_specs=pl.BlockSpec((tblk, od), lambda i: (i + off, 0)),
        out_shape=jax.ShapeDtypeStruct((n_total, od), jnp.float32),
        **kwargs,
    )(*ins)


def kernel(type_ids, op_ids, fine_ids, obj_hashes, text_hashes, field_masks,
           time_feats, type_W, op_W, fine_W, obj_W, text_W, mask_W, mask_b,
           time_W, time_b, W1, b1, g1, bln1, W2, b2, g2, bln2):
    B, L = type_ids.shape
    N = B * L
    tpw = N // _NW
    cpw = tpw // _CHUNK

    bt = B // _CHUNK
    ns = 2
    tidx = (text_hashes.astype(jnp.int32)
            .reshape(bt, _CHUNK, L, _F)
            .transpose(2, 0, 3, 1)
            .reshape(ns, _NW, _F * cpw // ns, _CHUNK))
    oidx = obj_hashes.astype(jnp.int32).T.reshape(ns, _NW, cpw // ns, _CHUNK)

    combs = [_sc_gather(text_W, obj_W, tidx[si], oidx[si], N // ns)
             for si in range(ns)]

    tblk = 2048
    g = N // tblk
    tids3 = type_ids.astype(jnp.int32).T.reshape(g, 1, tblk)
    oids3 = op_ids.astype(jnp.int32).T.reshape(g, 1, tblk)
    fids3 = fine_ids.astype(jnp.int32).T.reshape(g, 1, tblk)
    masks = (field_masks.astype(jnp.float32).transpose(1, 0, 2)
             .reshape(N, field_masks.shape[-1]))
    tfeat = time_feats.transpose(1, 0, 2).reshape(N, time_feats.shape[-1])
    g_full = jnp.concatenate([
        type_W @ W1[0:32], op_W @ W1[32:64], fine_W @ W1[64:96],
        W1[128:192], W1[96:128],
        mask_W @ W1[192:224], time_W @ W1[224:256]], axis=0)
    b1_eff = (b1 + mask_b @ W1[192:224] + time_b @ W1[224:256]).reshape(1, -1)
    weights = (g_full.astype(jnp.bfloat16), b1_eff,
               g1.reshape(1, -1), bln1.reshape(1, -1),
               W2.astype(jnp.bfloat16), b2.reshape(1, -1),
               g2.reshape(1, -1), bln2.reshape(1, -1))

    out = None
    for si in range(ns):
        out = _tc_mlp_slice(tids3, oids3, fids3, masks, tfeat, combs[si],
                            weights, tblk, N, si, out)
    return out.reshape(L, B, -1).transpose(1, 0, 2)
